# baseline (speedup 1.0000x reference)
import jax
import jax.numpy as jnp
from jax.experimental import pallas as pl
from jax.experimental.pallas import tpu as pltpu


def _cheb_kernel(data_ref, adj_ref, w_ref, b_ref, out_ref):
    adj = adj_ref[...]
    deg = jnp.sum(adj.astype(jnp.float32), axis=1, keepdims=True)
    dinv = jnp.where(deg > 0, deg ** -0.5, 0.0)

    x0 = data_ref[0]

    def mop(v):
        sv = (dinv * v).astype(jnp.bfloat16)
        u = jax.lax.dot_general(
            adj, sv, (((0,), (0,)), ((), ())),
            preferred_element_type=jnp.float32)
        return -dinv * u

    x1 = mop(x0)
    x2 = 2.0 * mop(x1) - x0

    acc = jnp.dot(x0.astype(jnp.bfloat16), w_ref[0],
                  preferred_element_type=jnp.float32)
    acc = acc + jnp.dot(x1.astype(jnp.bfloat16), w_ref[1],
                        preferred_element_type=jnp.float32)
    acc = acc + jnp.dot(x2.astype(jnp.bfloat16), w_ref[2],
                        preferred_element_type=jnp.float32)
    out_ref[0] = acc + b_ref[...]


def kernel(data, adj, W, b):
    B, N, F_IN = data.shape
    K, _, F_OUT = W.shape
    b2 = b.reshape(1, F_OUT)
    adj = adj.astype(jnp.bfloat16)
    W = W.astype(jnp.bfloat16)
    return pl.pallas_call(
        _cheb_kernel,
        grid=(B,),
        in_specs=[
            pl.BlockSpec((1, N, F_IN), lambda i: (i, 0, 0)),
            pl.BlockSpec((N, N), lambda i: (0, 0)),
            pl.BlockSpec((K, F_IN, F_OUT), lambda i: (0, 0, 0)),
            pl.BlockSpec((1, F_OUT), lambda i: (0, 0)),
        ],
        out_specs=pl.BlockSpec((1, N, F_OUT), lambda i: (i, 0, 0)),
        out_shape=jax.ShapeDtypeStruct((B, N, F_OUT), jnp.float32),
        compiler_params=pltpu.CompilerParams(
            dimension_semantics=("parallel",),
        ),
    )(data, adj, W, b2)


# speedup vs baseline: 1.0436x; 1.0436x over previous
import jax
import jax.numpy as jnp
from jax.experimental import pallas as pl
from jax.experimental.pallas import tpu as pltpu


def _prep_kernel(adj_ref, lt_ref):
    adj = adj_ref[...]
    deg = jnp.sum(adj, axis=1, keepdims=True)
    dinv = jnp.where(deg > 0, deg ** -0.5, 0.0)
    s = dinv * adj
    st = s.T
    lt_ref[...] = ((-dinv) * st).astype(jnp.bfloat16)


def _cheb_kernel(data_ref, lt_ref, w_ref, b_ref, out_ref):
    lt = lt_ref[...]
    x0f = data_ref[0]
    x0 = x0f.astype(jnp.bfloat16)
    x1f = jnp.dot(lt, x0, preferred_element_type=jnp.float32)
    x1 = x1f.astype(jnp.bfloat16)
    x2f = 2.0 * jnp.dot(lt, x1, preferred_element_type=jnp.float32) - x0f
    x2 = x2f.astype(jnp.bfloat16)
    acc = jnp.dot(x0, w_ref[0], preferred_element_type=jnp.float32)
    acc = acc + jnp.dot(x1, w_ref[1], preferred_element_type=jnp.float32)
    acc = acc + jnp.dot(x2, w_ref[2], preferred_element_type=jnp.float32)
    out_ref[0] = acc + b_ref[...]


def kernel(data, adj, W, b):
    B, N, F_IN = data.shape
    K, _, F_OUT = W.shape

    lhatT = pl.pallas_call(
        _prep_kernel,
        out_shape=jax.ShapeDtypeStruct((N, N), jnp.bfloat16),
    )(adj)

    return pl.pallas_call(
        _cheb_kernel,
        grid=(B,),
        in_specs=[
            pl.BlockSpec((1, N, F_IN), lambda i: (i, 0, 0)),
            pl.BlockSpec((N, N), lambda i: (0, 0)),
            pl.BlockSpec((K, F_IN, F_OUT), lambda i: (0, 0, 0)),
            pl.BlockSpec((1, F_OUT), lambda i: (0, 0)),
        ],
        out_specs=pl.BlockSpec((1, N, F_OUT), lambda i: (i, 0, 0)),
        out_shape=jax.ShapeDtypeStruct((B, N, F_OUT), jnp.float32),
        compiler_params=pltpu.CompilerParams(
            dimension_semantics=("parallel",),
        ),
    )(data, lhatT, W.astype(jnp.bfloat16), b.reshape(1, F_OUT))


# speedup vs baseline: 1.0476x; 1.0038x over previous
import jax
import jax.numpy as jnp
from jax.experimental import pallas as pl
from jax.experimental.pallas import tpu as pltpu


def _prep_kernel(adj_ref, lt_ref):
    adj = adj_ref[...]
    deg = jnp.sum(adj, axis=1, keepdims=True)
    dinv = jnp.where(deg > 0, deg ** -0.5, 0.0)
    s = dinv * adj
    st = s.T
    lt_ref[...] = ((-dinv) * st).astype(jnp.bfloat16)


def _cheb_kernel(data_ref, lt_ref, w_ref, b_ref, out_ref):
    lt = lt_ref[...]
    x0f = data_ref[0]
    x0 = x0f.astype(jnp.bfloat16)
    x1f = jnp.dot(lt, x0, preferred_element_type=jnp.float32)
    x1 = x1f.astype(jnp.bfloat16)
    x2f = 2.0 * jnp.dot(lt, x1, preferred_element_type=jnp.float32) - x0f
    x2 = x2f.astype(jnp.bfloat16)
    acc = jnp.dot(x0, w_ref[0], preferred_element_type=jnp.float32)
    acc = acc + jnp.dot(x1, w_ref[1], preferred_element_type=jnp.float32)
    acc = acc + jnp.dot(x2, w_ref[2], preferred_element_type=jnp.float32)
    out_ref[0] = acc + b_ref[...]


def kernel(data, adj, W, b):
    B, N, F_IN = data.shape
    K, _, F_OUT = W.shape

    lhatT = pl.pallas_call(
        _prep_kernel,
        out_shape=jax.ShapeDtypeStruct((N, N), jnp.bfloat16),
    )(adj)

    return pl.pallas_call(
        _cheb_kernel,
        grid=(B,),
        in_specs=[
            pl.BlockSpec((1, N, F_IN), lambda i: (i, 0, 0)),
            pl.BlockSpec((N, N), lambda i: (0, 0)),
            pl.BlockSpec((K, F_IN, F_OUT), lambda i: (0, 0, 0)),
            pl.BlockSpec((1, F_OUT), lambda i: (0, 0)),
        ],
        out_specs=pl.BlockSpec((1, N, F_OUT), lambda i: (i, 0, 0)),
        out_shape=jax.ShapeDtypeStruct((B, N, F_OUT), jnp.float32),
        compiler_params=pltpu.CompilerParams(
            dimension_semantics=("arbitrary",),
        ),
    )(data, lhatT, W.astype(jnp.bfloat16), b.reshape(1, F_OUT))


# speedup vs baseline: 1.1652x; 1.1122x over previous
import jax
import jax.numpy as jnp
from jax.experimental import pallas as pl
from jax.experimental.pallas import tpu as pltpu


def _cheb_kernel(adj_ref, data_ref, w_ref, b_ref, out_ref, lt_ref):
    @pl.when(pl.program_id(0) == 0)
    def _prep():
        adj = adj_ref[...]
        deg = jnp.sum(adj, axis=1, keepdims=True)
        dinv = jnp.where(deg > 0, deg ** -0.5, 0.0)
        s = dinv * adj
        lt_ref[...] = ((-dinv) * s.T).astype(jnp.bfloat16)

    lt = lt_ref[...]
    x0f = data_ref[0]
    x0 = x0f.astype(jnp.bfloat16)
    x1f = jnp.dot(lt, x0, preferred_element_type=jnp.float32)
    x1 = x1f.astype(jnp.bfloat16)
    x2f = 2.0 * jnp.dot(lt, x1, preferred_element_type=jnp.float32) - x0f
    x2 = x2f.astype(jnp.bfloat16)
    acc = jnp.dot(x0, w_ref[0], preferred_element_type=jnp.float32)
    acc = acc + jnp.dot(x1, w_ref[1], preferred_element_type=jnp.float32)
    acc = acc + jnp.dot(x2, w_ref[2], preferred_element_type=jnp.float32)
    out_ref[0] = acc + b_ref[...]


def kernel(data, adj, W, b):
    B, N, F_IN = data.shape
    K, _, F_OUT = W.shape

    return pl.pallas_call(
        _cheb_kernel,
        grid=(B,),
        in_specs=[
            pl.BlockSpec((N, N), lambda i: (0, 0)),
            pl.BlockSpec((1, N, F_IN), lambda i: (i, 0, 0)),
            pl.BlockSpec((K, F_IN, F_OUT), lambda i: (0, 0, 0)),
            pl.BlockSpec((1, F_OUT), lambda i: (0, 0)),
        ],
        out_specs=pl.BlockSpec((1, N, F_OUT), lambda i: (i, 0, 0)),
        out_shape=jax.ShapeDtypeStruct((B, N, F_OUT), jnp.float32),
        scratch_shapes=[pltpu.VMEM((N, N), jnp.bfloat16)],
        compiler_params=pltpu.CompilerParams(
            dimension_semantics=("arbitrary",),
        ),
    )(adj, data, W.astype(jnp.bfloat16), b.reshape(1, F_OUT))
